# baseline (speedup 1.0000x reference)
import functools

import numpy as np
import jax
import jax.numpy as jnp
from jax import lax
from jax.experimental import pallas as pl
from jax.experimental.pallas import tpu as pltpu
from jax.experimental.pallas import tpu_sc as plsc

N = 10000
E = 320000
D = 128
NC, NS = 2, 16
NW = NC * NS
NPAD = 10112
DEG_PAD = 10240
EPT = 10240
EPAD = EPT * NW
RPS = NPAD // NS
RPS_DEG = DEG_PAD // NS

CH = 64
NCHUNK = EPT // CH
NR = 4
NITER = NCHUNK // NR

DCH = 16
NDCH = 39
DREM = RPS - NDCH * DCH

CH_DEG = 128
NCHUNK_DEG = EPT // CH_DEG

TH0, TH1, TH2 = 0.6, -0.4, 0.2

_mesh = plsc.VectorSubcoreMesh(
    core_axis_name="c", subcore_axis_name="s", num_cores=NC, num_subcores=NS)


def _place_matrices():
    plo = np.zeros((D // 2, D), np.float32)
    phi = np.zeros((D // 2, D), np.float32)
    for k in range(D // 2):
        g, r = divmod(k, 16)
        plo[k, 32 * g + r] = 1.0
        phi[k, 32 * g + 16 + r] = 1.0
    return plo, phi


_PLO, _PHI = (jnp.asarray(m) for m in _place_matrices())


@functools.partial(
    pl.kernel,
    out_type=jax.ShapeDtypeStruct((NC, DEG_PAD), jnp.float32),
    mesh=_mesh,
    scratch_types=[
        pltpu.VMEM((CH_DEG,), jnp.int32),
        pltpu.VMEM((CH_DEG,), jnp.float32),
        pltpu.VMEM((RPS_DEG,), jnp.float32),
        pltpu.VMEM_SHARED((DEG_PAD,), jnp.float32),
    ],
)
def _deg_kernel(src_hbm, out_hbm, idx_v, ones_v, zb_v, hist_sh):
    c = lax.axis_index("c")
    s = lax.axis_index("s")
    wid = s * NC + c
    zeros16 = jnp.zeros((16,), jnp.float32)
    ones16 = jnp.ones((16,), jnp.float32)

    @pl.loop(0, CH_DEG // 16)
    def _(i):
        ones_v[pl.ds(i * 16, 16)] = ones16

    @pl.loop(0, RPS_DEG // 16)
    def _(i):
        zb_v[pl.ds(i * 16, 16)] = zeros16

    pltpu.sync_copy(zb_v, hist_sh.at[pl.ds(s * RPS_DEG, RPS_DEG)])
    plsc.subcore_barrier()
    base = wid * EPT

    @pl.loop(0, NCHUNK_DEG)
    def _(g):
        eb = base + g * CH_DEG
        pltpu.sync_copy(src_hbm.at[pl.ds(eb, CH_DEG)], idx_v)
        pltpu.sync_copy(ones_v, hist_sh.at[idx_v], add=True)

    plsc.subcore_barrier()
    pltpu.sync_copy(hist_sh.at[pl.ds(s * RPS_DEG, RPS_DEG)],
                    out_hbm.at[c, pl.ds(s * RPS_DEG, RPS_DEG)])


_round_scratch = (
    [
        pltpu.VMEM((DCH, D), jnp.float32),
        pltpu.VMEM_SHARED((NPAD, D), jnp.float32),
    ]
    + [pltpu.VMEM((DCH, D // 2), jnp.int32)] * 1
    + [pltpu.VMEM((2, CH), jnp.int32)] * (2 * NR)
    + [pltpu.VMEM((CH, D), jnp.float32)] * NR
    + [pltpu.SemaphoreType.DMA] * (4 * NR)
)


@functools.partial(
    pl.kernel,
    out_type=jax.ShapeDtypeStruct((NC, NPAD, D // 2), jnp.int32),
    mesh=_mesh,
    scratch_types=_round_scratch,
)
def _round_kernel(h_hbm, sd_hbm, out_hbm, zfd_v, agg_sh, *rest):
    zb16 = rest[0:1]
    sd = [rest[1:1 + NR], rest[1 + NR:1 + 2 * NR]]
    rows = rest[1 + 2 * NR:1 + 3 * NR]
    isem = [rest[1 + 3 * NR:1 + 4 * NR], rest[1 + 4 * NR:1 + 5 * NR]]
    gsem = rest[1 + 5 * NR:1 + 6 * NR]
    ssem = rest[1 + 6 * NR:1 + 7 * NR]

    c = lax.axis_index("c")
    s = lax.axis_index("s")
    wid = s * NC + (1 - c)
    zeros16 = jnp.zeros((16,), jnp.float32)

    @pl.loop(0, DCH)
    def _(i):
        for j in range(D // 16):
            zfd_v[i, pl.ds(j * 16, 16)] = zeros16

    @pl.loop(0, NDCH)
    def _(dc):
        pltpu.sync_copy(zfd_v, agg_sh.at[pl.ds(s * RPS + dc * DCH, DCH)])

    pltpu.sync_copy(zfd_v.at[pl.ds(0, DREM)],
                    agg_sh.at[pl.ds(s * RPS + NDCH * DCH, DREM)])
    plsc.subcore_barrier()

    cbase = wid * NCHUNK

    def start_idx(h, b, g):
        pltpu.async_copy(sd_hbm.at[cbase + g], sd[h][b], isem[h][b])

    def wait_idx(h, b):
        pltpu.make_async_copy(sd_hbm.at[0], sd[h][b], isem[h][b]).wait()

    def start_gather(h, b):
        pltpu.async_copy(h_hbm.at[sd[h][b].at[0]], rows[b], gsem[b])

    def wait_gather(h, b):
        pltpu.make_async_copy(h_hbm.at[sd[h][b].at[0]], rows[b],
                              gsem[b]).wait()

    def start_scatter(h, b):
        pltpu.async_copy(rows[b], agg_sh.at[sd[h][b].at[1]], ssem[b],
                         add=True)

    def wait_scatter(h, b):
        pltpu.make_async_copy(rows[b], agg_sh.at[sd[h][b].at[1]],
                              ssem[b]).wait()

    for b in range(NR):
        start_idx(0, b, b)

    @pl.loop(0, NITER // 2)
    def _(K):
        g0 = 2 * NR * K
        for b in range(NR):
            wait_idx(0, b)

            @pl.when(K > 0)
            def _():
                wait_scatter(1, b)

            start_gather(0, b)
        for b in range(NR):
            wait_gather(0, b)
            start_scatter(0, b)
            start_idx(1, b, g0 + NR + b)
        for b in range(NR):
            wait_idx(1, b)
            wait_scatter(0, b)
            start_gather(1, b)
        for b in range(NR):
            wait_gather(1, b)
            start_scatter(1, b)

            @pl.when(K < NITER // 2 - 1)
            def _():
                start_idx(0, b, g0 + 2 * NR + b)

    for b in range(NR):
        wait_scatter(1, b)
    plsc.subcore_barrier()

    def to_bf16_bits(x):
        u = lax.bitcast_convert_type(x, jnp.uint32)
        return (u + jnp.uint32(0x7FFF) + ((u >> 16) & jnp.uint32(1))) >> 16

    def pack_chunk(nrows):
        @pl.loop(0, nrows)
        def _(r):
            for grp in range(D // 32):
                a = zfd_v[r, pl.ds(grp * 32, 16)]
                b = zfd_v[r, pl.ds(grp * 32 + 16, 16)]
                word = to_bf16_bits(a) | (to_bf16_bits(b) << 16)
                zb16[0][r, pl.ds(grp * 16, 16)] = lax.bitcast_convert_type(
                    word, jnp.int32)

    @pl.loop(0, NDCH)
    def _(dc):
        row0 = s * RPS + dc * DCH
        pltpu.sync_copy(agg_sh.at[pl.ds(row0, DCH)], zfd_v)
        pack_chunk(DCH)
        pltpu.sync_copy(zb16[0], out_hbm.at[c, pl.ds(row0, DCH)])

    rowr = s * RPS + NDCH * DCH
    pltpu.sync_copy(agg_sh.at[pl.ds(rowr, DREM)], zfd_v.at[pl.ds(0, DREM)])
    pack_chunk(DREM)
    pltpu.sync_copy(zb16[0].at[pl.ds(0, DREM)],
                    out_hbm.at[c, pl.ds(rowr, DREM)])


def _prep_body(degp_ref, dinv_ref):
    deg = degp_ref[0] + degp_ref[1]
    dinv_ref[...] = jax.lax.rsqrt(jnp.maximum(deg, 1.0))


def _prep_kernel(degp):
    return pl.pallas_call(
        _prep_body,
        out_shape=jax.ShapeDtypeStruct((DEG_PAD // D, D), jnp.float32),
    )(degp.reshape(NC, DEG_PAD // D, D))


def _scale_body(feat_ref, dinv_ref, h_ref, outp_ref):
    h_ref[...] = feat_ref[...] * dinv_ref[...]
    outp_ref[...] = TH0 * feat_ref[...]


def _scale_kernel(feat, dinv_col):
    blk = 1000
    return pl.pallas_call(
        _scale_body,
        grid=(N // blk,),
        in_specs=[
            pl.BlockSpec((blk, D), lambda i: (i, 0)),
            pl.BlockSpec((blk, 1), lambda i: (i, 0)),
        ],
        out_specs=[pl.BlockSpec((blk, D), lambda i: (i, 0))] * 2,
        out_shape=[jax.ShapeDtypeStruct((N, D), jnp.float32)] * 2,
    )(feat, dinv_col)


def _agg_nat(aggp_ref, plo_ref, phi_ref):
    u0 = lax.bitcast_convert_type(aggp_ref[0], jnp.uint32)
    u1 = lax.bitcast_convert_type(aggp_ref[1], jnp.uint32)

    def lo(u):
        return lax.bitcast_convert_type((u & jnp.uint32(0xFFFF)) << 16,
                                        jnp.float32)

    def hi(u):
        return lax.bitcast_convert_type(u & jnp.uint32(0xFFFF0000),
                                        jnp.float32)

    los = lo(u0) + lo(u1)
    his = hi(u0) + hi(u1)
    return (jnp.dot(los, plo_ref[...], preferred_element_type=jnp.float32)
            + jnp.dot(his, phi_ref[...], preferred_element_type=jnp.float32))


def _comb_body(aggp_ref, f_ref, outp_ref, dinv_ref, th_ref, plo_ref, phi_ref,
               fn_ref, hn_ref, outn_ref):
    dinv = dinv_ref[...]
    agg = _agg_nat(aggp_ref, plo_ref, phi_ref)
    f_next = f_ref[...] - agg * dinv
    fn_ref[...] = f_next
    hn_ref[...] = f_next * dinv
    outn_ref[...] = outp_ref[...] + th_ref[0, 0] * f_next


def _comb_kernel(aggp, f_prev, outp, dinv_col, th):
    blk = 1000
    shp = jax.ShapeDtypeStruct((N, D), jnp.float32)
    return pl.pallas_call(
        _comb_body,
        grid=(N // blk,),
        in_specs=[
            pl.BlockSpec((NC, blk, D // 2), lambda i: (0, i, 0)),
            pl.BlockSpec((blk, D), lambda i: (i, 0)),
            pl.BlockSpec((blk, D), lambda i: (i, 0)),
            pl.BlockSpec((blk, 1), lambda i: (i, 0)),
            pl.BlockSpec((1, 1), lambda i: (0, 0)),
            pl.BlockSpec((D // 2, D), lambda i: (0, 0)),
            pl.BlockSpec((D // 2, D), lambda i: (0, 0)),
        ],
        out_specs=[pl.BlockSpec((blk, D), lambda i: (i, 0))] * 3,
        out_shape=[shp, shp, shp],
    )(aggp, f_prev, outp, dinv_col, th, _PLO, _PHI)


def kernel(feat, edge_index):
    src = edge_index[0]
    dst = edge_index[1]
    pad = EPAD - E
    src_d = jnp.concatenate([src, jnp.full((pad,), N, jnp.int32)])
    src_g = jnp.concatenate([src, jnp.zeros((pad,), jnp.int32)])
    dst_p = jnp.concatenate([dst, jnp.full((pad,), N, jnp.int32)])
    sd = jnp.stack([src_g.reshape(NW * NCHUNK, CH),
                    dst_p.reshape(NW * NCHUNK, CH)], axis=1)

    degp = _deg_kernel(src_d)
    dinv = _prep_kernel(degp)
    dinv_col = dinv.reshape(DEG_PAD)[:N][:, None]

    h1, outp0 = _scale_kernel(feat, dinv_col)

    def body(carry, th):
        f_prev, h, outp = carry
        aggp = _round_kernel(h, sd)
        f_next, h_next, outp_next = _comb_kernel(aggp, f_prev, outp,
                                                 dinv_col, th)
        return (f_next, h_next, outp_next), None

    thetas = jnp.array([TH1, TH2], jnp.float32).reshape(2, 1, 1)
    (_, _, out), _ = lax.scan(body, (feat, h1, outp0), thetas)
    return out


# speedup vs baseline: 1.0572x; 1.0572x over previous
import functools

import numpy as np
import jax
import jax.numpy as jnp
from jax import lax
from jax.experimental import pallas as pl
from jax.experimental.pallas import tpu as pltpu
from jax.experimental.pallas import tpu_sc as plsc

N = 10000
E = 320000
D = 128
NC, NS = 2, 16
NW = NC * NS
NPAD = 10112
DEG_PAD = 10240
EPT = 10240
EPAD = EPT * NW
RPS = NPAD // NS
RPS_DEG = DEG_PAD // NS

CH = 64
NCHUNK = EPT // CH
NR = 4
NITER = NCHUNK // NR

DCH = 16
NDCH = 39
DREM = RPS - NDCH * DCH

CH_DEG = 128
NCHUNK_DEG = EPT // CH_DEG

TH0, TH1, TH2 = 0.6, -0.4, 0.2

_mesh = plsc.VectorSubcoreMesh(
    core_axis_name="c", subcore_axis_name="s", num_cores=NC, num_subcores=NS)


def _place_matrices():
    plo = np.zeros((D // 2, D), np.float32)
    phi = np.zeros((D // 2, D), np.float32)
    for k in range(D // 2):
        g, r = divmod(k, 16)
        plo[k, 32 * g + r] = 1.0
        phi[k, 32 * g + 16 + r] = 1.0
    return plo, phi


_PLO, _PHI = (jnp.asarray(m) for m in _place_matrices())


@functools.partial(
    pl.kernel,
    out_type=jax.ShapeDtypeStruct((NC, DEG_PAD), jnp.float32),
    mesh=_mesh,
    scratch_types=[
        pltpu.VMEM((CH_DEG,), jnp.int32),
        pltpu.VMEM((CH_DEG,), jnp.float32),
        pltpu.VMEM((RPS_DEG,), jnp.float32),
        pltpu.VMEM_SHARED((DEG_PAD,), jnp.float32),
    ],
)
def _deg_kernel(src_hbm, out_hbm, idx_v, ones_v, zb_v, hist_sh):
    c = lax.axis_index("c")
    s = lax.axis_index("s")
    wid = s * NC + c
    zeros16 = jnp.zeros((16,), jnp.float32)
    ones16 = jnp.ones((16,), jnp.float32)

    @pl.loop(0, CH_DEG // 16)
    def _(i):
        ones_v[pl.ds(i * 16, 16)] = ones16

    @pl.loop(0, RPS_DEG // 16)
    def _(i):
        zb_v[pl.ds(i * 16, 16)] = zeros16

    pltpu.sync_copy(zb_v, hist_sh.at[pl.ds(s * RPS_DEG, RPS_DEG)])
    plsc.subcore_barrier()
    base = wid * EPT

    @pl.loop(0, NCHUNK_DEG)
    def _(g):
        eb = base + g * CH_DEG
        pltpu.sync_copy(src_hbm.at[pl.ds(eb, CH_DEG)], idx_v)
        pltpu.sync_copy(ones_v, hist_sh.at[idx_v], add=True)

    plsc.subcore_barrier()
    pltpu.sync_copy(hist_sh.at[pl.ds(s * RPS_DEG, RPS_DEG)],
                    out_hbm.at[c, pl.ds(s * RPS_DEG, RPS_DEG)])


_round_scratch = (
    [
        pltpu.VMEM((DCH, D), jnp.float32),
        pltpu.VMEM_SHARED((NPAD, D), jnp.float32),
    ]
    + [pltpu.VMEM((DCH, D // 2), jnp.int32)] * 1
    + [pltpu.VMEM((2, CH), jnp.int32)] * (2 * NR)
    + [pltpu.VMEM((CH, D), jnp.float32)] * NR
    + [pltpu.SemaphoreType.DMA] * (4 * NR)
)


@functools.partial(
    pl.kernel,
    out_type=jax.ShapeDtypeStruct((NC, NPAD, D // 2), jnp.int32),
    mesh=_mesh,
    scratch_types=_round_scratch,
)
def _round_kernel(h_hbm, sd_hbm, out_hbm, zfd_v, agg_sh, *rest):
    zb16 = rest[0:1]
    sd = [rest[1:1 + NR], rest[1 + NR:1 + 2 * NR]]
    rows = rest[1 + 2 * NR:1 + 3 * NR]
    isem = [rest[1 + 3 * NR:1 + 4 * NR], rest[1 + 4 * NR:1 + 5 * NR]]
    gsem = rest[1 + 5 * NR:1 + 6 * NR]
    ssem = rest[1 + 6 * NR:1 + 7 * NR]

    c = lax.axis_index("c")
    s = lax.axis_index("s")
    wid = s * NC + c
    zeros16 = jnp.zeros((16,), jnp.float32)

    @pl.loop(0, DCH)
    def _(i):
        for j in range(D // 16):
            zfd_v[i, pl.ds(j * 16, 16)] = zeros16

    @pl.loop(0, NDCH)
    def _(dc):
        pltpu.sync_copy(zfd_v, agg_sh.at[pl.ds(s * RPS + dc * DCH, DCH)])

    pltpu.sync_copy(zfd_v.at[pl.ds(0, DREM)],
                    agg_sh.at[pl.ds(s * RPS + NDCH * DCH, DREM)])
    plsc.subcore_barrier()

    cbase = wid * NCHUNK

    def start_idx(h, b, g):
        pltpu.async_copy(sd_hbm.at[cbase + g], sd[h][b], isem[h][b])

    def wait_idx(h, b):
        pltpu.make_async_copy(sd_hbm.at[0], sd[h][b], isem[h][b]).wait()

    def start_gather(h, b):
        pltpu.async_copy(h_hbm.at[sd[h][b].at[0]], rows[b], gsem[b])

    def wait_gather(h, b):
        pltpu.make_async_copy(h_hbm.at[sd[h][b].at[0]], rows[b],
                              gsem[b]).wait()

    def start_scatter(h, b):
        pltpu.async_copy(rows[b], agg_sh.at[sd[h][b].at[1]], ssem[b],
                         add=True)

    def wait_scatter(h, b):
        pltpu.make_async_copy(rows[b], agg_sh.at[sd[h][b].at[1]],
                              ssem[b]).wait()

    for b in range(NR):
        start_idx(0, b, b)

    @pl.loop(0, NITER // 2)
    def _(K):
        g0 = 2 * NR * K
        for b in range(NR):
            wait_idx(0, b)

            @pl.when(K > 0)
            def _():
                wait_scatter(1, b)

            start_gather(0, b)
        for b in range(NR):
            wait_gather(0, b)
            start_scatter(0, b)
            start_idx(1, b, g0 + NR + b)
        for b in range(NR):
            wait_idx(1, b)
            wait_scatter(0, b)
            start_gather(1, b)
        for b in range(NR):
            wait_gather(1, b)
            start_scatter(1, b)

            @pl.when(K < NITER // 2 - 1)
            def _():
                start_idx(0, b, g0 + 2 * NR + b)

    for b in range(NR):
        wait_scatter(1, b)
    plsc.subcore_barrier()

    def to_bf16_bits(x):
        u = lax.bitcast_convert_type(x, jnp.uint32)
        return (u + jnp.uint32(0x7FFF) + ((u >> 16) & jnp.uint32(1))) >> 16

    def pack_chunk(nrows):
        @pl.loop(0, nrows)
        def _(r):
            for grp in range(D // 32):
                a = zfd_v[r, pl.ds(grp * 32, 16)]
                b = zfd_v[r, pl.ds(grp * 32 + 16, 16)]
                word = to_bf16_bits(a) | (to_bf16_bits(b) << 16)
                zb16[0][r, pl.ds(grp * 16, 16)] = lax.bitcast_convert_type(
                    word, jnp.int32)

    @pl.loop(0, NDCH)
    def _(dc):
        row0 = s * RPS + dc * DCH
        pltpu.sync_copy(agg_sh.at[pl.ds(row0, DCH)], zfd_v)
        pack_chunk(DCH)
        pltpu.sync_copy(zb16[0], out_hbm.at[c, pl.ds(row0, DCH)])

    rowr = s * RPS + NDCH * DCH
    pltpu.sync_copy(agg_sh.at[pl.ds(rowr, DREM)], zfd_v.at[pl.ds(0, DREM)])
    pack_chunk(DREM)
    pltpu.sync_copy(zb16[0].at[pl.ds(0, DREM)],
                    out_hbm.at[c, pl.ds(rowr, DREM)])


def _prep_body(degp_ref, dinv_ref):
    deg = degp_ref[0] + degp_ref[1]
    dinv_ref[...] = jax.lax.rsqrt(jnp.maximum(deg, 1.0))


def _prep_kernel(degp):
    return pl.pallas_call(
        _prep_body,
        out_shape=jax.ShapeDtypeStruct((DEG_PAD // D, D), jnp.float32),
    )(degp.reshape(NC, DEG_PAD // D, D))


def _scale_body(feat_ref, dinv_ref, h_ref, outp_ref):
    h_ref[...] = feat_ref[...] * dinv_ref[...]
    outp_ref[...] = TH0 * feat_ref[...]


def _scale_kernel(feat, dinv_col):
    blk = 1000
    return pl.pallas_call(
        _scale_body,
        grid=(N // blk,),
        in_specs=[
            pl.BlockSpec((blk, D), lambda i: (i, 0)),
            pl.BlockSpec((blk, 1), lambda i: (i, 0)),
        ],
        out_specs=[pl.BlockSpec((blk, D), lambda i: (i, 0))] * 2,
        out_shape=[jax.ShapeDtypeStruct((N, D), jnp.float32)] * 2,
    )(feat, dinv_col)


def _agg_nat(aggp_ref, plo_ref, phi_ref):
    u0 = lax.bitcast_convert_type(aggp_ref[0], jnp.uint32)
    u1 = lax.bitcast_convert_type(aggp_ref[1], jnp.uint32)

    def lo(u):
        return lax.bitcast_convert_type((u & jnp.uint32(0xFFFF)) << 16,
                                        jnp.float32)

    def hi(u):
        return lax.bitcast_convert_type(u & jnp.uint32(0xFFFF0000),
                                        jnp.float32)

    los = lo(u0) + lo(u1)
    his = hi(u0) + hi(u1)
    return (jnp.dot(los, plo_ref[...], preferred_element_type=jnp.float32)
            + jnp.dot(his, phi_ref[...], preferred_element_type=jnp.float32))


def _comb_body(aggp_ref, f_ref, outp_ref, dinv_ref, th_ref, plo_ref, phi_ref,
               fn_ref, hn_ref, outn_ref):
    dinv = dinv_ref[...]
    agg = _agg_nat(aggp_ref, plo_ref, phi_ref)
    f_next = f_ref[...] - agg * dinv
    fn_ref[...] = f_next
    hn_ref[...] = f_next * dinv
    outn_ref[...] = outp_ref[...] + th_ref[0, 0] * f_next


def _comb_kernel(aggp, f_prev, outp, dinv_col, th):
    blk = 1000
    shp = jax.ShapeDtypeStruct((N, D), jnp.float32)
    return pl.pallas_call(
        _comb_body,
        grid=(N // blk,),
        in_specs=[
            pl.BlockSpec((NC, blk, D // 2), lambda i: (0, i, 0)),
            pl.BlockSpec((blk, D), lambda i: (i, 0)),
            pl.BlockSpec((blk, D), lambda i: (i, 0)),
            pl.BlockSpec((blk, 1), lambda i: (i, 0)),
            pl.BlockSpec((1, 1), lambda i: (0, 0)),
            pl.BlockSpec((D // 2, D), lambda i: (0, 0)),
            pl.BlockSpec((D // 2, D), lambda i: (0, 0)),
        ],
        out_specs=[pl.BlockSpec((blk, D), lambda i: (i, 0))] * 3,
        out_shape=[shp, shp, shp],
    )(aggp, f_prev, outp, dinv_col, th, _PLO, _PHI)


def kernel(feat, edge_index):
    src = edge_index[0]
    dst = edge_index[1]
    pad = EPAD - E
    src_d = jnp.concatenate([src, jnp.full((pad,), N, jnp.int32)])
    src_g = jnp.concatenate([src, jnp.zeros((pad,), jnp.int32)])
    dst_p = jnp.concatenate(
        [dst, N + (jnp.arange(pad, dtype=jnp.int32) % (NPAD - N))])
    sd = jnp.stack([src_g.reshape(NW * NCHUNK, CH),
                    dst_p.reshape(NW * NCHUNK, CH)], axis=1)

    degp = _deg_kernel(src_d)
    dinv = _prep_kernel(degp)
    dinv_col = dinv.reshape(DEG_PAD)[:N][:, None]

    h1, outp0 = _scale_kernel(feat, dinv_col)

    def body(carry, th):
        f_prev, h, outp = carry
        aggp = _round_kernel(h, sd)
        f_next, h_next, outp_next = _comb_kernel(aggp, f_prev, outp,
                                                 dinv_col, th)
        return (f_next, h_next, outp_next), None

    thetas = jnp.array([TH1, TH2], jnp.float32).reshape(2, 1, 1)
    (_, _, out), _ = lax.scan(body, (feat, h1, outp0), thetas)
    return out


# speedup vs baseline: 1.1286x; 1.0675x over previous
import functools

import numpy as np
import jax
import jax.numpy as jnp
from jax import lax
from jax.experimental import pallas as pl
from jax.experimental.pallas import tpu as pltpu
from jax.experimental.pallas import tpu_sc as plsc

N = 10000
E = 320000
D = 128
NC, NS = 2, 16
NW = NC * NS
NPAD = 10112
DEG_PAD = 10240
EPT = 10240
EPAD = EPT * NW
RPS = NPAD // NS
RPS_DEG = DEG_PAD // NS

CH = 64
NCHUNK = EPT // CH
NR = 4
NITER = NCHUNK // NR

EPT_A = 15872
EPT_B = 2 * EPT - EPT_A
NIT_A = EPT_A // CH // (2 * NR)
NIT_B = EPT_B // CH // (2 * NR)

DCH = 16
NDCH = 39
DREM = RPS - NDCH * DCH

CH_DEG = 128
NCHUNK_DEG = EPT // CH_DEG

TH0, TH1, TH2 = 0.6, -0.4, 0.2

_mesh = plsc.VectorSubcoreMesh(
    core_axis_name="c", subcore_axis_name="s", num_cores=NC, num_subcores=NS)


def _place_matrices():
    plo = np.zeros((D // 2, D), np.float32)
    phi = np.zeros((D // 2, D), np.float32)
    for k in range(D // 2):
        g, r = divmod(k, 16)
        plo[k, 32 * g + r] = 1.0
        phi[k, 32 * g + 16 + r] = 1.0
    return plo, phi


_PLO, _PHI = (jnp.asarray(m) for m in _place_matrices())


@functools.partial(
    pl.kernel,
    out_type=jax.ShapeDtypeStruct((NC, DEG_PAD), jnp.float32),
    mesh=_mesh,
    scratch_types=[
        pltpu.VMEM((CH_DEG,), jnp.int32),
        pltpu.VMEM((CH_DEG,), jnp.float32),
        pltpu.VMEM((RPS_DEG,), jnp.float32),
        pltpu.VMEM_SHARED((DEG_PAD,), jnp.float32),
    ],
)
def _deg_kernel(src_hbm, out_hbm, idx_v, ones_v, zb_v, hist_sh):
    c = lax.axis_index("c")
    s = lax.axis_index("s")
    wid = s * NC + c
    zeros16 = jnp.zeros((16,), jnp.float32)
    ones16 = jnp.ones((16,), jnp.float32)

    @pl.loop(0, CH_DEG // 16)
    def _(i):
        ones_v[pl.ds(i * 16, 16)] = ones16

    @pl.loop(0, RPS_DEG // 16)
    def _(i):
        zb_v[pl.ds(i * 16, 16)] = zeros16

    pltpu.sync_copy(zb_v, hist_sh.at[pl.ds(s * RPS_DEG, RPS_DEG)])
    plsc.subcore_barrier()
    base = wid * EPT

    @pl.loop(0, NCHUNK_DEG)
    def _(g):
        eb = base + g * CH_DEG
        pltpu.sync_copy(src_hbm.at[pl.ds(eb, CH_DEG)], idx_v)
        pltpu.sync_copy(ones_v, hist_sh.at[idx_v], add=True)

    plsc.subcore_barrier()
    pltpu.sync_copy(hist_sh.at[pl.ds(s * RPS_DEG, RPS_DEG)],
                    out_hbm.at[c, pl.ds(s * RPS_DEG, RPS_DEG)])


_round_scratch = (
    [
        pltpu.VMEM((DCH, D), jnp.float32),
        pltpu.VMEM_SHARED((NPAD, D), jnp.float32),
    ]
    + [pltpu.VMEM((DCH, D // 2), jnp.int32)] * 1
    + [pltpu.VMEM((2, CH), jnp.int32)] * (2 * NR)
    + [pltpu.VMEM((CH, D), jnp.float32)] * NR
    + [pltpu.SemaphoreType.DMA] * (4 * NR)
)


@functools.partial(
    pl.kernel,
    out_type=jax.ShapeDtypeStruct((NC, NPAD, D // 2), jnp.int32),
    mesh=_mesh,
    scratch_types=_round_scratch,
)
def _round_kernel(h_hbm, sd_hbm, out_hbm, zfd_v, agg_sh, *rest):
    zb16 = rest[0:1]
    sd = [rest[1:1 + NR], rest[1 + NR:1 + 2 * NR]]
    rows = rest[1 + 2 * NR:1 + 3 * NR]
    isem = [rest[1 + 3 * NR:1 + 4 * NR], rest[1 + 4 * NR:1 + 5 * NR]]
    gsem = rest[1 + 5 * NR:1 + 6 * NR]
    ssem = rest[1 + 6 * NR:1 + 7 * NR]

    c = lax.axis_index("c")
    s = lax.axis_index("s")
    wid = s * NC + c
    zeros16 = jnp.zeros((16,), jnp.float32)

    @pl.loop(0, DCH)
    def _(i):
        for j in range(D // 16):
            zfd_v[i, pl.ds(j * 16, 16)] = zeros16

    @pl.loop(0, NDCH)
    def _(dc):
        pltpu.sync_copy(zfd_v, agg_sh.at[pl.ds(s * RPS + dc * DCH, DCH)])

    pltpu.sync_copy(zfd_v.at[pl.ds(0, DREM)],
                    agg_sh.at[pl.ds(s * RPS + NDCH * DCH, DREM)])
    plsc.subcore_barrier()

    ebase = jnp.where(c == 0, s * EPT_A, NS * EPT_A + s * EPT_B)
    cbase = ebase // CH
    niter = jnp.where(c == 0, NIT_A, NIT_B)

    def start_idx(h, b, g):
        pltpu.async_copy(sd_hbm.at[cbase + g], sd[h][b], isem[h][b])

    def wait_idx(h, b):
        pltpu.make_async_copy(sd_hbm.at[0], sd[h][b], isem[h][b]).wait()

    def start_gather(h, b):
        pltpu.async_copy(h_hbm.at[sd[h][b].at[0]], rows[b], gsem[b])

    def wait_gather(h, b):
        pltpu.make_async_copy(h_hbm.at[sd[h][b].at[0]], rows[b],
                              gsem[b]).wait()

    def start_scatter(h, b):
        pltpu.async_copy(rows[b], agg_sh.at[sd[h][b].at[1]], ssem[b],
                         add=True)

    def wait_scatter(h, b):
        pltpu.make_async_copy(rows[b], agg_sh.at[sd[h][b].at[1]],
                              ssem[b]).wait()

    for b in range(NR):
        start_idx(0, b, b)

    @pl.loop(0, niter)
    def _(K):
        g0 = 2 * NR * K
        for b in range(NR):
            wait_idx(0, b)

            @pl.when(K > 0)
            def _():
                wait_scatter(1, b)

            start_gather(0, b)
        for b in range(NR):
            wait_gather(0, b)
            start_scatter(0, b)
            start_idx(1, b, g0 + NR + b)
        for b in range(NR):
            wait_idx(1, b)
            wait_scatter(0, b)
            start_gather(1, b)
        for b in range(NR):
            wait_gather(1, b)
            start_scatter(1, b)

            @pl.when(K < niter - 1)
            def _():
                start_idx(0, b, g0 + 2 * NR + b)

    for b in range(NR):
        wait_scatter(1, b)
    plsc.subcore_barrier()

    def to_bf16_bits(x):
        u = lax.bitcast_convert_type(x, jnp.uint32)
        return (u + jnp.uint32(0x7FFF) + ((u >> 16) & jnp.uint32(1))) >> 16

    def pack_chunk(nrows):
        @pl.loop(0, nrows)
        def _(r):
            for grp in range(D // 32):
                a = zfd_v[r, pl.ds(grp * 32, 16)]
                b = zfd_v[r, pl.ds(grp * 32 + 16, 16)]
                word = to_bf16_bits(a) | (to_bf16_bits(b) << 16)
                zb16[0][r, pl.ds(grp * 16, 16)] = lax.bitcast_convert_type(
                    word, jnp.int32)

    @pl.loop(0, NDCH)
    def _(dc):
        row0 = s * RPS + dc * DCH
        pltpu.sync_copy(agg_sh.at[pl.ds(row0, DCH)], zfd_v)
        pack_chunk(DCH)
        pltpu.sync_copy(zb16[0], out_hbm.at[c, pl.ds(row0, DCH)])

    rowr = s * RPS + NDCH * DCH
    pltpu.sync_copy(agg_sh.at[pl.ds(rowr, DREM)], zfd_v.at[pl.ds(0, DREM)])
    pack_chunk(DREM)
    pltpu.sync_copy(zb16[0].at[pl.ds(0, DREM)],
                    out_hbm.at[c, pl.ds(rowr, DREM)])


def _prep_body(degp_ref, dinv_ref):
    deg = degp_ref[0] + degp_ref[1]
    dinv_ref[...] = jax.lax.rsqrt(jnp.maximum(deg, 1.0))


def _prep_kernel(degp):
    return pl.pallas_call(
        _prep_body,
        out_shape=jax.ShapeDtypeStruct((DEG_PAD // D, D), jnp.float32),
    )(degp.reshape(NC, DEG_PAD // D, D))


def _scale_body(feat_ref, dinv_ref, h_ref, outp_ref):
    h_ref[...] = feat_ref[...] * dinv_ref[...]
    outp_ref[...] = TH0 * feat_ref[...]


def _scale_kernel(feat, dinv_col):
    blk = 1000
    return pl.pallas_call(
        _scale_body,
        grid=(N // blk,),
        in_specs=[
            pl.BlockSpec((blk, D), lambda i: (i, 0)),
            pl.BlockSpec((blk, 1), lambda i: (i, 0)),
        ],
        out_specs=[pl.BlockSpec((blk, D), lambda i: (i, 0))] * 2,
        out_shape=[jax.ShapeDtypeStruct((N, D), jnp.float32)] * 2,
    )(feat, dinv_col)


def _agg_nat(aggp_ref, plo_ref, phi_ref):
    u0 = lax.bitcast_convert_type(aggp_ref[0], jnp.uint32)
    u1 = lax.bitcast_convert_type(aggp_ref[1], jnp.uint32)

    def lo(u):
        return lax.bitcast_convert_type((u & jnp.uint32(0xFFFF)) << 16,
                                        jnp.float32)

    def hi(u):
        return lax.bitcast_convert_type(u & jnp.uint32(0xFFFF0000),
                                        jnp.float32)

    los = lo(u0) + lo(u1)
    his = hi(u0) + hi(u1)
    return (jnp.dot(los, plo_ref[...], preferred_element_type=jnp.float32)
            + jnp.dot(his, phi_ref[...], preferred_element_type=jnp.float32))


def _comb_body(aggp_ref, f_ref, outp_ref, dinv_ref, th_ref, plo_ref, phi_ref,
               fn_ref, hn_ref, outn_ref):
    dinv = dinv_ref[...]
    agg = _agg_nat(aggp_ref, plo_ref, phi_ref)
    f_next = f_ref[...] - agg * dinv
    fn_ref[...] = f_next
    hn_ref[...] = f_next * dinv
    outn_ref[...] = outp_ref[...] + th_ref[0, 0] * f_next


def _comb_kernel(aggp, f_prev, outp, dinv_col, th):
    blk = 1000
    shp = jax.ShapeDtypeStruct((N, D), jnp.float32)
    return pl.pallas_call(
        _comb_body,
        grid=(N // blk,),
        in_specs=[
            pl.BlockSpec((NC, blk, D // 2), lambda i: (0, i, 0)),
            pl.BlockSpec((blk, D), lambda i: (i, 0)),
            pl.BlockSpec((blk, D), lambda i: (i, 0)),
            pl.BlockSpec((blk, 1), lambda i: (i, 0)),
            pl.BlockSpec((1, 1), lambda i: (0, 0)),
            pl.BlockSpec((D // 2, D), lambda i: (0, 0)),
            pl.BlockSpec((D // 2, D), lambda i: (0, 0)),
        ],
        out_specs=[pl.BlockSpec((blk, D), lambda i: (i, 0))] * 3,
        out_shape=[shp, shp, shp],
    )(aggp, f_prev, outp, dinv_col, th, _PLO, _PHI)


def kernel(feat, edge_index):
    src = edge_index[0]
    dst = edge_index[1]
    pad = EPAD - E
    src_d = jnp.concatenate([src, jnp.full((pad,), N, jnp.int32)])
    src_g = jnp.concatenate([src, jnp.zeros((pad,), jnp.int32)])
    dst_p = jnp.concatenate(
        [dst, N + (jnp.arange(pad, dtype=jnp.int32) % (NPAD - N))])
    sd = jnp.stack([src_g.reshape(NW * NCHUNK, CH),
                    dst_p.reshape(NW * NCHUNK, CH)], axis=1)

    degp = _deg_kernel(src_d)
    dinv = _prep_kernel(degp)
    dinv_col = dinv.reshape(DEG_PAD)[:N][:, None]

    h1, outp0 = _scale_kernel(feat, dinv_col)

    def body(carry, th):
        f_prev, h, outp = carry
        aggp = _round_kernel(h, sd)
        f_next, h_next, outp_next = _comb_kernel(aggp, f_prev, outp,
                                                 dinv_col, th)
        return (f_next, h_next, outp_next), None

    thetas = jnp.array([TH1, TH2], jnp.float32).reshape(2, 1, 1)
    (_, _, out), _ = lax.scan(body, (feat, h1, outp0), thetas)
    return out
